# Initial kernel scaffold; baseline (speedup 1.0000x reference)
import jax
import jax.numpy as jnp
from jax.experimental import pallas as pl


def kernel(node_feat, adj, convW, convB, mlpW, mlpB, lnG, lnB, linW, linB):
    raise NotImplementedError("write your pallas kernel here")


# speedup vs baseline: 1.4552x; 1.4552x over previous
import jax
import jax.numpy as jnp
from jax.experimental import pallas as pl

_N = 2048
_HID = 128
_OUT = 64
_NL = 2


def _fused_gcn_kernel(x_ref, adj_ref, convW_ref, convB_ref, mlpW_ref,
                      mlpB_ref, lnG_ref, lnB_ref, linW_ref, linB_ref,
                      out_ref):
    f32 = jnp.float32
    A = adj_ref[...]
    rows = jax.lax.broadcasted_iota(jnp.int32, (_N, _N), 0)
    cols = jax.lax.broadcasted_iota(jnp.int32, (_N, _N), 1)
    diag = rows == cols
    W = jnp.where(A > 0, A, jnp.where(diag, f32(1.0), f32(0.0)))
    deg = jnp.sum(W, axis=0, keepdims=True)
    dinv_row = jax.lax.rsqrt(deg)
    dinv_col = dinv_row.reshape(_N, 1)

    x = x_ref[...]
    for l in range(_NL):
        h = jnp.dot(x, convW_ref[l], preferred_element_type=f32)
        hs = dinv_col * h
        agg = jax.lax.dot_general(W, hs, (((0,), (0,)), ((), ())),
                                  preferred_element_type=f32)
        x = dinv_col * agg + convB_ref[l][None, :]
        x = jnp.dot(x, mlpW_ref[l], preferred_element_type=f32)
        x = x + mlpB_ref[l][None, :]
        mu = jnp.mean(x, axis=-1, keepdims=True)
        var = jnp.mean((x - mu) ** 2, axis=-1, keepdims=True)
        x = (x - mu) * jax.lax.rsqrt(var + f32(1e-5))
        x = x * lnG_ref[l][None, :] + lnB_ref[l][None, :]
        x = jnp.maximum(x, f32(0.0))
    out_ref[...] = jnp.dot(x, linW_ref[...], preferred_element_type=f32) + \
        linB_ref[...][None, :]


def kernel(node_feat, adj, convW, convB, mlpW, mlpB, lnG, lnB, linW, linB):
    x2d = node_feat[0]
    adj2d = adj[0]
    out = pl.pallas_call(
        _fused_gcn_kernel,
        out_shape=jax.ShapeDtypeStruct((_N, _OUT), jnp.float32),
    )(x2d, adj2d, convW, convB, mlpW, mlpB, lnG, lnB, linW, linB)
    return out[None]
